# baseline (speedup 1.0000x reference)
import jax
import jax.numpy as jnp
import numpy as np
from jax.experimental import pallas as pl
from jax.experimental.pallas import tpu as pltpu

N_BINS = 15
_BOUNDS = np.linspace(0.0, 1.0, N_BINS + 1, dtype=np.float32)
_LOWERS = _BOUNDS[:-1]
_UPPERS = _BOUNDS[1:]


def _ece_kernel(x_ref, conf_ref, lab_ref, bounds_ref, out_ref, acc_ref):
    i = pl.program_id(0)
    nb = pl.num_programs(0)

    @pl.when(i == 0)
    def _init():
        acc_ref[...] = jnp.zeros_like(acc_ref)

    x = x_ref[...]
    block, c = x.shape
    m = jnp.max(x, axis=1, keepdims=True)
    col = jax.lax.broadcasted_iota(jnp.int32, (block, c), 1)
    ji = jnp.where(x == m, col, c)
    first_max = jnp.min(ji, axis=1, keepdims=True)
    acc = (first_max == lab_ref[...]).astype(jnp.float32)

    conf = conf_ref[...]
    lowers = bounds_ref[0:1, :]
    uppers = bounds_ref[1:2, :]
    mask = ((conf > lowers) & (conf <= uppers)).astype(jnp.float32)

    cnt = jnp.sum(mask, axis=0, keepdims=True)
    sumc = jnp.sum(mask * conf, axis=0, keepdims=True)
    suma = jnp.sum(mask * acc, axis=0, keepdims=True)
    acc_ref[...] += jnp.concatenate([cnt, sumc, suma], axis=0)

    @pl.when(i == nb - 1)
    def _finish():
        tot = acc_ref[0:1, :]
        sc = acc_ref[1:2, :]
        sa = acc_ref[2:3, :]
        n = block * nb
        safe = jnp.where(tot > 0, tot, 1.0)
        contrib = jnp.where(
            tot > 0,
            jnp.abs(sc / safe - sa / safe) * (tot / n),
            0.0,
        )
        out_ref[...] = jnp.sum(contrib).reshape(1, 1)


def kernel(softmaxes, confidences, labels):
    n, c = softmaxes.shape
    block = 1000
    nb = n // block
    conf2 = confidences.reshape(n, 1)
    lab2 = labels.astype(jnp.int32).reshape(n, 1)
    bounds = jnp.asarray(np.stack([_LOWERS, _UPPERS]))

    out = pl.pallas_call(
        _ece_kernel,
        grid=(nb,),
        in_specs=[
            pl.BlockSpec((block, c), lambda i: (i, 0)),
            pl.BlockSpec((block, 1), lambda i: (i, 0)),
            pl.BlockSpec((block, 1), lambda i: (i, 0)),
            pl.BlockSpec((2, N_BINS), lambda i: (0, 0)),
        ],
        out_specs=pl.BlockSpec((1, 1), lambda i: (0, 0)),
        out_shape=jax.ShapeDtypeStruct((1, 1), jnp.float32),
        scratch_shapes=[pltpu.VMEM((3, N_BINS), jnp.float32)],
    )(softmaxes, conf2, lab2, bounds)
    return out.reshape(1)


# speedup vs baseline: 3.7224x; 3.7224x over previous
import jax
import jax.numpy as jnp
import numpy as np
from jax.experimental import pallas as pl
from jax.experimental.pallas import tpu as pltpu

N_BINS = 15
_BOUNDS = np.linspace(0.0, 1.0, N_BINS + 1, dtype=np.float32)
_CB = 40
_SUB = 8


def _ece_kernel(x_ref, conf_ref, lab_ref, out_ref, m_ref, b_ref):
    i = pl.program_id(0)
    nb = pl.num_programs(0)

    @pl.when(i == 0)
    def _init():
        m_ref[...] = jnp.full_like(m_ref, -jnp.inf)
        b_ref[...] = jnp.zeros_like(b_ref)

    m = m_ref[...]
    b = b_ref[...]
    for j in range(_CB // _SUB):
        sub = x_ref[_SUB * j:_SUB * (j + 1), :]
        upd = sub > m
        base = i * _CB + j * _SUB
        m = jnp.where(upd, sub, m)
        b = jnp.where(upd, base, b)
    m_ref[...] = m
    b_ref[...] = b

    @pl.when(i == nb - 1)
    def _finish():
        mm = m_ref[...]
        idx = b_ref[...] + jax.lax.broadcasted_iota(jnp.int32, mm.shape, 0)
        gmax = jnp.max(mm, axis=0, keepdims=True)
        ji = jnp.where(mm == gmax, idx, jnp.int32(1 << 30))
        fmi = jnp.min(ji, axis=0, keepdims=True)
        acc = (fmi == lab_ref[...]).astype(jnp.float32)

        conf = conf_ref[...]
        n = conf.shape[1]
        ece = jnp.zeros((1, 1), jnp.float32)
        for k in range(N_BINS):
            lo = float(_BOUNDS[k])
            hi = float(_BOUNDS[k + 1])
            mask = ((conf > lo) & (conf <= hi)).astype(jnp.float32)
            cnt = jnp.sum(mask).reshape(1, 1)
            sumc = jnp.sum(mask * conf).reshape(1, 1)
            suma = jnp.sum(mask * acc).reshape(1, 1)
            safe = jnp.where(cnt > 0, cnt, 1.0)
            ece += jnp.where(
                cnt > 0,
                jnp.abs(sumc / safe - suma / safe) * (cnt / n),
                0.0,
            )
        out_ref[...] = ece


def kernel(softmaxes, confidences, labels):
    n, c = softmaxes.shape
    xt = softmaxes.T
    nb = c // _CB
    conf2 = confidences.reshape(1, n)
    lab2 = labels.astype(jnp.int32).reshape(1, n)

    out = pl.pallas_call(
        _ece_kernel,
        grid=(nb,),
        in_specs=[
            pl.BlockSpec((_CB, n), lambda i: (i, 0)),
            pl.BlockSpec((1, n), lambda i: (0, 0)),
            pl.BlockSpec((1, n), lambda i: (0, 0)),
        ],
        out_specs=pl.BlockSpec((1, 1), lambda i: (0, 0)),
        out_shape=jax.ShapeDtypeStruct((1, 1), jnp.float32),
        scratch_shapes=[
            pltpu.VMEM((_SUB, n), jnp.float32),
            pltpu.VMEM((_SUB, n), jnp.int32),
        ],
    )(xt, conf2, lab2)
    return out.reshape(1)


# speedup vs baseline: 4.0993x; 1.1013x over previous
import jax
import jax.numpy as jnp
import numpy as np
from jax.experimental import pallas as pl
from jax.experimental.pallas import tpu as pltpu

N_BINS = 15
_BOUNDS = np.linspace(0.0, 1.0, N_BINS + 1, dtype=np.float32)
_CB = 40
_SUB = 8


def _ece_kernel(x_ref, conf_ref, lab_ref, bounds_ref, out_ref, m_ref, b_ref):
    i = pl.program_id(0)
    nb = pl.num_programs(0)

    @pl.when(i == 0)
    def _init():
        m_ref[...] = jnp.full_like(m_ref, -jnp.inf)
        b_ref[...] = jnp.zeros_like(b_ref)

    m = m_ref[...]
    b = b_ref[...]
    for j in range(_CB // _SUB):
        sub = x_ref[_SUB * j:_SUB * (j + 1), :]
        upd = sub > m
        base = i * _CB + j * _SUB
        m = jnp.where(upd, sub, m)
        b = jnp.where(upd, base, b)
    m_ref[...] = m
    b_ref[...] = b

    @pl.when(i == nb - 1)
    def _finish():
        mm = m_ref[...]
        idx = b_ref[...] + jax.lax.broadcasted_iota(jnp.int32, mm.shape, 0)
        gmax = jnp.max(mm, axis=0, keepdims=True)
        ji = jnp.where(mm == gmax, idx, jnp.int32(1 << 30))
        fmi = jnp.min(ji, axis=0, keepdims=True)
        acc = (fmi == lab_ref[...]).astype(jnp.float32)

        conf = conf_ref[...]
        n = conf.shape[1]
        conf_b = jnp.broadcast_to(conf, (_SUB, n))
        acc_b = jnp.broadcast_to(acc, (_SUB, n))
        ece = jnp.zeros((1, 1), jnp.float32)
        for g in range(2):
            lob = bounds_ref[_SUB * g:_SUB * (g + 1), 0:1]
            hib = bounds_ref[_SUB * g:_SUB * (g + 1), 1:2]
            mask = ((conf_b > lob) & (conf_b <= hib)).astype(jnp.float32)
            cnt = jnp.sum(mask, axis=1, keepdims=True)
            sumc = jnp.sum(mask * conf_b, axis=1, keepdims=True)
            suma = jnp.sum(mask * acc_b, axis=1, keepdims=True)
            safe = jnp.where(cnt > 0, cnt, 1.0)
            contrib = jnp.where(
                cnt > 0,
                jnp.abs(sumc / safe - suma / safe) * (cnt / n),
                0.0,
            )
            ece += jnp.sum(contrib).reshape(1, 1)
        out_ref[...] = ece


def kernel(softmaxes, confidences, labels):
    n, c = softmaxes.shape
    xt = softmaxes.T
    nb = c // _CB
    conf2 = confidences.reshape(1, n)
    lab2 = labels.astype(jnp.int32).reshape(1, n)
    bnp = np.full((16, 2), 2.0, dtype=np.float32)
    bnp[:N_BINS, 0] = _BOUNDS[:-1]
    bnp[:N_BINS, 1] = _BOUNDS[1:]
    bounds = jnp.asarray(bnp)

    out = pl.pallas_call(
        _ece_kernel,
        grid=(nb,),
        in_specs=[
            pl.BlockSpec((_CB, n), lambda i: (i, 0)),
            pl.BlockSpec((1, n), lambda i: (0, 0)),
            pl.BlockSpec((1, n), lambda i: (0, 0)),
            pl.BlockSpec((16, 2), lambda i: (0, 0)),
        ],
        out_specs=pl.BlockSpec((1, 1), lambda i: (0, 0)),
        out_shape=jax.ShapeDtypeStruct((1, 1), jnp.float32),
        scratch_shapes=[
            pltpu.VMEM((_SUB, n), jnp.float32),
            pltpu.VMEM((_SUB, n), jnp.int32),
        ],
    )(xt, conf2, lab2, bounds)
    return out.reshape(1)
